# baseline (speedup 1.0000x reference)
import functools

import jax
import jax.numpy as jnp
from jax import lax
from jax.experimental import pallas as pl
from jax.experimental.pallas import tpu as pltpu
from jax.experimental.pallas import tpu_sc as plsc

B = 16384
D = 64
NC = 2
NS = 16
NW = NC * NS
BPW = B // NW
C = 16
NCH = BPW // C
CR = C * 8


def _gmf_body(uid_hbm, iid_hbm, ut_hbm, it_hbm, w_hbm, b_hbm, out_hbm,
              idx_u, idx_i, tu_a, tu_b, ti_a, ti_b, w_v, b_v, out_v,
              sem_u, sem_i):
    wid = lax.axis_index("s") * NC + lax.axis_index("c")
    base = wid * BPW

    pltpu.sync_copy(uid_hbm.at[pl.ds(base, BPW)], idx_u)
    pltpu.sync_copy(iid_hbm.at[pl.ds(base, BPW)], idx_i)
    pltpu.sync_copy(w_hbm, w_v)
    pltpu.sync_copy(b_hbm, b_v)

    w0 = w_v[pl.ds(0, 16)]
    w1 = w_v[pl.ds(16, 16)]
    w2 = w_v[pl.ds(32, 16)]
    w3 = w_v[pl.ds(48, 16)]
    bvec = b_v[...]
    lane = lax.iota(jnp.int32, 16)

    def issue(ch, bu, bi):
        uvec = idx_u[pl.ds(ch * C, 16)] & ~7
        ivec = idx_i[pl.ds(ch * C, 16)] & ~7
        for k in range(16):
            su = pl.multiple_of(uvec[k], 8)
            si = pl.multiple_of(ivec[k], 8)
            pltpu.async_copy(ut_hbm.at[pl.ds(su, 8)],
                             bu.at[pl.ds(k * 8, 8)], sem_u)
            pltpu.async_copy(it_hbm.at[pl.ds(si, 8)],
                             bi.at[pl.ds(k * 8, 8)], sem_i)

    def drain():
        pltpu.make_async_copy(ut_hbm.at[pl.ds(0, CR)], tu_a, sem_u).wait()
        pltpu.make_async_copy(it_hbm.at[pl.ds(0, CR)], ti_a, sem_i).wait()

    def compute(ch, bu, bi):
        ru_vec = idx_u[pl.ds(ch * C, 16)] & 7
        ri_vec = idx_i[pl.ds(ch * C, 16)] & 7
        vec = jnp.zeros((16,), jnp.float32)
        for k in range(16):
            ru = k * 8 + ru_vec[k]
            ri = k * 8 + ri_vec[k]
            acc = bu[ru, pl.ds(0, 16)] * bi[ri, pl.ds(0, 16)] * w0
            acc += bu[ru, pl.ds(16, 16)] * bi[ri, pl.ds(16, 16)] * w1
            acc += bu[ru, pl.ds(32, 16)] * bi[ri, pl.ds(32, 16)] * w2
            acc += bu[ru, pl.ds(48, 16)] * bi[ri, pl.ds(48, 16)] * w3
            vec = jnp.where(lane == k, jnp.sum(acc), vec)
        out_v[pl.ds(ch * C, 16)] = vec + bvec

    issue(0, tu_a, ti_a)

    def super_chunk(s, carry):
        ch = 2 * s
        issue(ch + 1, tu_b, ti_b)
        drain()
        compute(ch, tu_a, ti_a)
        issue((ch + 2) % NCH, tu_a, ti_a)
        drain()
        compute(ch + 1, tu_b, ti_b)
        return carry

    lax.fori_loop(0, NCH // 2, super_chunk, 0)
    drain()

    pltpu.sync_copy(out_v, out_hbm.at[pl.ds(base, BPW)])


@jax.jit
def kernel(userID, itemID, user_table, item_table, W, b):
    w1d = W.reshape(D)
    b16 = jnp.broadcast_to(b.astype(jnp.float32), (16,))

    mesh = plsc.VectorSubcoreMesh(core_axis_name="c", subcore_axis_name="s")
    f = pl.kernel(
        _gmf_body,
        mesh=mesh,
        compiler_params=pltpu.CompilerParams(needs_layout_passes=False),
        out_type=jax.ShapeDtypeStruct((B,), jnp.float32),
        scratch_types=[
            pltpu.VMEM((BPW,), jnp.int32),
            pltpu.VMEM((BPW,), jnp.int32),
            pltpu.VMEM((CR, D), jnp.float32),
            pltpu.VMEM((CR, D), jnp.float32),
            pltpu.VMEM((CR, D), jnp.float32),
            pltpu.VMEM((CR, D), jnp.float32),
            pltpu.VMEM((D,), jnp.float32),
            pltpu.VMEM((16,), jnp.float32),
            pltpu.VMEM((BPW,), jnp.float32),
            pltpu.SemaphoreType.DMA,
            pltpu.SemaphoreType.DMA,
        ],
    )
    return f(userID, itemID, user_table, item_table, w1d, b16)
